# Initial kernel scaffold; baseline (speedup 1.0000x reference)
import jax
import jax.numpy as jnp
from jax.experimental import pallas as pl


def kernel(x, edge_index, edge_attr, W_tconv, b_tconv, W_msg, b_msg, W_self, b_self, W_asrc, b_asrc, W_adst, b_adst):
    raise NotImplementedError("write your pallas kernel here")


# speedup vs baseline: 17.5142x; 17.5142x over previous
import functools

import jax
import jax.numpy as jnp
from jax import lax
from jax.experimental import pallas as pl
from jax.experimental.pallas import tpu as pltpu
from jax.experimental.pallas import tpu_sc as plsc

N = 10000
T = 4
C = 128
K = 3
E = 160000

NC = 2
NS = 16
L = 16
C2 = C // NC

NB1 = 2048
G1 = -(-N // NB1)
NB2 = N // T
G2 = T
NPG = N // T

E_PER = E // NS
CH = 80
NCH = E_PER // CH
ROWS_PER = 632
NROW = NS * ROWS_PER
NPAD = NS * 640


def _tc1_body(x_ref, wt_ref, wm_ref, wsrc_ref, wdst_ref, bt_ref, bm_ref,
              bias_ref, xm_ref, asrc_ref, adst_ref, amax_ref):
    i = pl.program_id(0)
    node_id = i * NB1 + jax.lax.broadcasted_iota(jnp.int32, (NB1, 1), 0)
    row_ok = node_id < N
    grp = node_id // NPG
    ninf = jnp.float32(-jnp.inf)
    a_s_cols = []
    a_d_cols = []
    for t in range(T):
        acc = None
        for k in range(K):
            tt = t + k - 1
            if 0 <= tt < T:
                d = jnp.dot(x_ref[:, tt, :], wt_ref[k],
                            preferred_element_type=jnp.float32)
                acc = d if acc is None else acc + d
        acc = acc + bt_ref[...]
        xm_t = jnp.dot(acc, wm_ref[...],
                       preferred_element_type=jnp.float32) + bm_ref[...]
        xm_ref[0, :, t, :] = xm_t[:, :C2]
        xm_ref[1, :, t, :] = xm_t[:, C2:]
        a_s = jnp.sum(acc * wsrc_ref[...], axis=1)
        a_d = jnp.sum(acc * wdst_ref[...], axis=1) + bias_ref[...][0]
        a_s_cols.append(a_s.reshape(NB1, 1))
        a_d_cols.append(a_d.reshape(NB1, 1))
    asrc_ref[...] = jnp.concatenate(a_s_cols, axis=1)
    adst_ref[...] = jnp.concatenate(a_d_cols, axis=1)
    gmaxes = []
    for g in range(T):
        mask = (grp == g) & row_ok
        gm = None
        for col in a_s_cols:
            m = jnp.max(jnp.where(mask, col, ninf))
            gm = m if gm is None else jnp.maximum(gm, m)
        gmaxes.append(gm.reshape(1, 1))
    cur = jnp.concatenate(gmaxes, axis=1)
    prev = jnp.where(i == 0, jnp.full((1, T), ninf, jnp.float32),
                     amax_ref[...])
    amax_ref[...] = jnp.maximum(prev, cur)


def _tc2_body(x_ref, agg_ref, ws_ref, bs_ref, out_ref):
    xf = x_ref[...].reshape(NB2 * T, C)
    res = jnp.dot(xf, ws_ref[...],
                  preferred_element_type=jnp.float32) + bs_ref[...]
    res3 = res.reshape(NB2, T, C)
    a0 = agg_ref[0, 0][:N].reshape(NB2, T, C2)
    a1 = agg_ref[1, 0][:N].reshape(NB2, T, C2)
    for t in range(T):
        aggt = jnp.concatenate([a0[:, t, :], a1[:, t, :]], axis=1)
        out_ref[:, t, :] = res3[:, t, :] + aggt


def _sc_body(src_hbm, dst_hbm, ea_hbm, asrc_hbm, adst_hbm, amax_hbm, xm_hbm,
             out_hbm, srcoff_v, dst_v, ea_v, asrc_v, adst_v, w_v, den_v,
             rows_v, dstc_v, amax_v, den_sh, agg_sh, sem):
    cid = lax.axis_index("c")
    sid = lax.axis_index("s")
    elo = sid * E_PER
    nlo = sid * ROWS_PER
    dlo = sid * 640

    pltpu.sync_copy(dst_hbm.at[pl.ds(elo, E_PER)], dst_v)
    pltpu.sync_copy(ea_hbm.at[pl.ds(elo, E_PER)], ea_v)
    pltpu.sync_copy(amax_hbm, amax_v)

    zero16 = jnp.zeros((L,), jnp.float32)

    for t in range(T):
        t_n = t * N

        pltpu.sync_copy(src_hbm.at[pl.ds(elo, E_PER)], srcoff_v)
        pltpu.sync_copy(asrc_hbm.at[pl.ds(t_n, N)], asrc_v)
        pltpu.sync_copy(adst_hbm.at[pl.ds(t_n, N)], adst_v)

        def zrow(r, _):
            for j in range(C2 // L):
                rows_v[r, pl.ds(j * L, L)] = zero16
            return 0
        lax.fori_loop(0, CH, zrow, 0)

        def zden(v, _):
            den_v[pl.ds(pl.multiple_of(v * L, L), L)] = zero16
            return 0
        lax.fori_loop(0, 640 // L, zden, 0)

        pltpu.sync_copy(den_v.at[pl.ds(0, 640)], den_sh.at[pl.ds(dlo, 640)])
        for cc in range(ROWS_PER // CH):
            pltpu.sync_copy(rows_v, agg_sh.at[pl.ds(nlo + cc * CH, CH)])
        rem = ROWS_PER % CH
        if rem:
            pltpu.sync_copy(
                rows_v.at[pl.ds(0, rem)],
                agg_sh.at[pl.ds(nlo + (ROWS_PER // CH) * CH, rem)])

        av = amax_v[pl.ds(0, L)]
        sel = jnp.where(lax.iota(jnp.int32, L) == t, av,
                        jnp.float32(-jnp.inf))
        amax_t = jnp.zeros((L,), jnp.float32) + jnp.max(sel)

        def p1(i, _):
            off = pl.multiple_of(i * L, L)
            s = srcoff_v[pl.ds(off, L)]
            d = dst_v[pl.ds(off, L)]
            a_s = plsc.load_gather(asrc_v, [s])
            a_d = plsc.load_gather(adst_v, [d])
            z = a_s + a_d
            lg = jnp.where(z >= 0.0, z, 0.2 * z)
            b = amax_t + a_d
            m = jnp.where(b >= 0.0, b, 0.2 * b)
            w_v[pl.ds(off, L)] = jnp.exp(lg - m)
            srcoff_v[pl.ds(off, L)] = s + t_n
            return 0
        lax.fori_loop(0, E_PER // L, p1, 0)

        plsc.subcore_barrier()

        def dchunk(c, _):
            base = pl.multiple_of(c * CH, CH)
            for v in range(CH // L):
                dstc_v[pl.ds(v * L, L)] = dst_v[pl.ds(base + v * L, L)]
            pltpu.sync_copy(w_v.at[pl.ds(base, CH)], den_sh.at[dstc_v],
                            add=True)
            return 0
        lax.fori_loop(0, NCH, dchunk, 0)

        plsc.subcore_barrier()

        pltpu.sync_copy(den_sh, den_v)

        def p2(i, _):
            off = pl.multiple_of(i * L, L)
            d = dst_v[pl.ds(off, L)]
            dn = plsc.load_gather(den_v, [d])
            w_v[pl.ds(off, L)] = (w_v[pl.ds(off, L)] * ea_v[pl.ds(off, L)]
                                  / (dn + 1e-16))
            return 0
        lax.fori_loop(0, E_PER // L, p2, 0)

        xmh = xm_hbm.at[cid]

        def chunk(c, _):
            base = pl.multiple_of(c * CH, CH)
            for v in range(CH // L):
                dstc_v[pl.ds(v * L, L)] = dst_v[pl.ds(base + v * L, L)]
            pltpu.async_copy(xmh.at[srcoff_v.at[pl.ds(base, CH)]],
                             rows_v, sem).wait()

            def srow(r, _):
                cs = plsc.load_gather(
                    w_v, [jnp.zeros((L,), jnp.int32) + (base + r)])
                for j in range(C2 // L):
                    rows_v[r, pl.ds(j * L, L)] = (
                        rows_v[r, pl.ds(j * L, L)] * cs)
                return 0
            lax.fori_loop(0, CH, srow, 0)

            pltpu.sync_copy(rows_v, agg_sh.at[dstc_v], add=True)
            return 0
        lax.fori_loop(0, NCH, chunk, 0)

        plsc.subcore_barrier()

        pltpu.sync_copy(agg_sh.at[pl.ds(nlo, ROWS_PER)],
                        out_hbm.at[cid].at[t].at[pl.ds(nlo, ROWS_PER)])


def _make_sc_kernel():
  return functools.partial(
    pl.kernel,
    out_type=jax.ShapeDtypeStruct((NC, T, NROW, C2), jnp.float32),
    mesh=plsc.VectorSubcoreMesh(core_axis_name="c", subcore_axis_name="s",
                                num_cores=NC, num_subcores=NS),
    compiler_params=pltpu.CompilerParams(needs_layout_passes=False,
                                         use_tc_tiling_on_sc=False),
    scratch_types=[
        pltpu.VMEM((E_PER,), jnp.int32),
        pltpu.VMEM((E_PER,), jnp.int32),
        pltpu.VMEM((E_PER,), jnp.float32),
        pltpu.VMEM((N,), jnp.float32),
        pltpu.VMEM((N,), jnp.float32),
        pltpu.VMEM((E_PER,), jnp.float32),
        pltpu.VMEM((NPAD,), jnp.float32),
        pltpu.VMEM((CH, C2), jnp.float32),
        pltpu.VMEM((CH,), jnp.int32),
        pltpu.VMEM((L,), jnp.float32),
        pltpu.VMEM_SHARED((NPAD,), jnp.float32),
        pltpu.VMEM_SHARED((NROW, C2), jnp.float32),
        pltpu.SemaphoreType.DMA,
    ],
  )(_sc_body)


@jax.jit
def kernel(x, edge_index, edge_attr, W_tconv, b_tconv, W_msg, b_msg,
           W_self, b_self, W_asrc, b_asrc, W_adst, b_adst):
    x = x.astype(jnp.float32)
    src = edge_index[0].astype(jnp.int32)
    dst = edge_index[1].astype(jnp.int32)
    ea = edge_attr[:, 0].astype(jnp.float32)

    wt = jnp.transpose(W_tconv, (2, 1, 0))
    wm = W_msg.T
    ws = W_self.T
    bt = b_tconv.reshape(1, C)
    bm = b_msg.reshape(1, C)
    bs = b_self.reshape(1, C)
    bias = (b_asrc + b_adst).reshape(1, 1)

    xm, asrc, adst, amax = pl.pallas_call(
        _tc1_body,
        grid=(G1,),
        in_specs=[
            pl.BlockSpec((NB1, T, C), lambda i: (i, 0, 0)),
            pl.BlockSpec((K, C, C), lambda i: (0, 0, 0)),
            pl.BlockSpec((C, C), lambda i: (0, 0)),
            pl.BlockSpec((1, C), lambda i: (0, 0)),
            pl.BlockSpec((1, C), lambda i: (0, 0)),
            pl.BlockSpec((1, C), lambda i: (0, 0)),
            pl.BlockSpec((1, C), lambda i: (0, 0)),
            pl.BlockSpec((1, 1), lambda i: (0, 0)),
        ],
        out_specs=[
            pl.BlockSpec((NC, NB1, T, C2), lambda i: (0, i, 0, 0)),
            pl.BlockSpec((NB1, T), lambda i: (i, 0)),
            pl.BlockSpec((NB1, T), lambda i: (i, 0)),
            pl.BlockSpec((1, T), lambda i: (0, 0)),
        ],
        out_shape=[
            jax.ShapeDtypeStruct((NC, N, T, C2), jnp.float32),
            jax.ShapeDtypeStruct((N, T), jnp.float32),
            jax.ShapeDtypeStruct((N, T), jnp.float32),
            jax.ShapeDtypeStruct((1, T), jnp.float32),
        ],
    )(x, wt, wm, W_asrc, W_adst, bt, bm, bias)

    amax16 = jnp.pad(amax.reshape(T), (0, L - T))
    xmflat = xm.reshape(NC, N * T, C2)
    asrcflat = asrc.reshape(N * T)
    adstflat = adst.reshape(N * T)

    agg = _make_sc_kernel()(src, dst, ea, asrcflat, adstflat, amax16, xmflat)

    out = pl.pallas_call(
        _tc2_body,
        grid=(G2,),
        in_specs=[
            pl.BlockSpec((NB2, T, C), lambda i: (i, 0, 0)),
            pl.BlockSpec((NC, 1, NROW, C2), lambda i: (0, i, 0, 0)),
            pl.BlockSpec((C, C), lambda i: (0, 0)),
            pl.BlockSpec((1, C), lambda i: (0, 0)),
        ],
        out_specs=pl.BlockSpec((NB2, T, C), lambda i: (i, 0, 0)),
        out_shape=jax.ShapeDtypeStruct((N, T, C), jnp.float32),
        compiler_params=pltpu.CompilerParams(
            vmem_limit_bytes=64 * 1024 * 1024),
    )(x, agg, ws, bs)

    return out


# speedup vs baseline: 26.1007x; 1.4903x over previous
import functools

import jax
import jax.numpy as jnp
from jax import lax
from jax.experimental import pallas as pl
from jax.experimental.pallas import tpu as pltpu
from jax.experimental.pallas import tpu_sc as plsc

N = 10000
T = 4
C = 128
K = 3
E = 160000

NC = 2
NS = 16
L = 16
C2 = C // NC

NB1 = 2048
G1 = -(-N // NB1)
NB2 = N // T
G2 = T
NPG = N // T

E_PER = E // NS
CH = 80
NCH = E_PER // CH
ROWS_PER = 632
NROW = NS * ROWS_PER
NPAD = NS * 640


def _tc1_body(x_ref, wt_ref, wm_ref, wsrc_ref, wdst_ref, bt_ref, bm_ref,
              bias_ref, xm_ref, asrc_ref, adst_ref, amax_ref):
    i = pl.program_id(0)
    node_id = i * NB1 + jax.lax.broadcasted_iota(jnp.int32, (NB1, 1), 0)
    row_ok = node_id < N
    grp = node_id // NPG
    ninf = jnp.float32(-jnp.inf)
    a_s_cols = []
    a_d_cols = []
    for t in range(T):
        acc = None
        for k in range(K):
            tt = t + k - 1
            if 0 <= tt < T:
                d = jnp.dot(x_ref[:, tt, :], wt_ref[k],
                            preferred_element_type=jnp.float32)
                acc = d if acc is None else acc + d
        acc = acc + bt_ref[...]
        xm_t = jnp.dot(acc, wm_ref[...],
                       preferred_element_type=jnp.float32) + bm_ref[...]
        xm_ref[0, :, t, :] = xm_t[:, :C2]
        xm_ref[1, :, t, :] = xm_t[:, C2:]
        a_s = jnp.sum(acc * wsrc_ref[...], axis=1)
        a_d = jnp.sum(acc * wdst_ref[...], axis=1) + bias_ref[...][0]
        a_s_cols.append(a_s.reshape(NB1, 1))
        a_d_cols.append(a_d.reshape(NB1, 1))
    asrc_ref[...] = jnp.concatenate(a_s_cols, axis=1)
    adst_ref[...] = jnp.concatenate(a_d_cols, axis=1)
    gmaxes = []
    for g in range(T):
        mask = (grp == g) & row_ok
        gm = None
        for col in a_s_cols:
            m = jnp.max(jnp.where(mask, col, ninf))
            gm = m if gm is None else jnp.maximum(gm, m)
        gmaxes.append(gm.reshape(1, 1))
    cur = jnp.concatenate(gmaxes, axis=1)
    prev = jnp.where(i == 0, jnp.full((1, T), ninf, jnp.float32),
                     amax_ref[...])
    amax_ref[...] = jnp.maximum(prev, cur)


def _tc2_body(x_ref, agg_ref, ws_ref, bs_ref, out_ref):
    xf = x_ref[...].reshape(NB2 * T, C)
    res = jnp.dot(xf, ws_ref[...],
                  preferred_element_type=jnp.float32) + bs_ref[...]
    res3 = res.reshape(NB2, T, C)
    a0 = agg_ref[0, 0][:N].reshape(NB2, T, C2)
    a1 = agg_ref[1, 0][:N].reshape(NB2, T, C2)
    for t in range(T):
        aggt = jnp.concatenate([a0[:, t, :], a1[:, t, :]], axis=1)
        out_ref[:, t, :] = res3[:, t, :] + aggt


def _sc_body(src_hbm, dst_hbm, ea_hbm, asrc_hbm, adst_hbm, amax_hbm, xm_hbm,
             out_hbm, srcoff_v, dst_v, ea_v, asrc_v, adst_v, w_v, den_v,
             rows_a, rows_b, dstc_a, dstc_b, amax_v, den_sh, agg_sh,
             sem_a, sem_b):
    rows_v = rows_a
    dstc_v = dstc_a
    sem = sem_a
    cid = lax.axis_index("c")
    sid = lax.axis_index("s")
    elo = sid * E_PER
    nlo = sid * ROWS_PER
    dlo = sid * 640

    pltpu.sync_copy(dst_hbm.at[pl.ds(elo, E_PER)], dst_v)
    pltpu.sync_copy(ea_hbm.at[pl.ds(elo, E_PER)], ea_v)
    pltpu.sync_copy(amax_hbm, amax_v)

    zero16 = jnp.zeros((L,), jnp.float32)

    for t in range(T):
        t_n = t * N

        pltpu.sync_copy(src_hbm.at[pl.ds(elo, E_PER)], srcoff_v)
        pltpu.sync_copy(asrc_hbm.at[pl.ds(t_n, N)], asrc_v)
        pltpu.sync_copy(adst_hbm.at[pl.ds(t_n, N)], adst_v)

        def zrow(r, _):
            for j in range(C2 // L):
                rows_v[r, pl.ds(j * L, L)] = zero16
            return 0
        lax.fori_loop(0, CH, zrow, 0)

        def zden(v, _):
            den_v[pl.ds(pl.multiple_of(v * L, L), L)] = zero16
            return 0
        lax.fori_loop(0, 640 // L, zden, 0)

        pltpu.sync_copy(den_v.at[pl.ds(0, 640)], den_sh.at[pl.ds(dlo, 640)])
        for cc in range(ROWS_PER // CH):
            pltpu.sync_copy(rows_v, agg_sh.at[pl.ds(nlo + cc * CH, CH)])
        rem = ROWS_PER % CH
        if rem:
            pltpu.sync_copy(
                rows_v.at[pl.ds(0, rem)],
                agg_sh.at[pl.ds(nlo + (ROWS_PER // CH) * CH, rem)])

        av = amax_v[pl.ds(0, L)]
        sel = jnp.where(lax.iota(jnp.int32, L) == t, av,
                        jnp.float32(-jnp.inf))
        amax_t = jnp.zeros((L,), jnp.float32) + jnp.max(sel)

        def p1(i, _):
            off = pl.multiple_of(i * L, L)
            s = srcoff_v[pl.ds(off, L)]
            d = dst_v[pl.ds(off, L)]
            a_s = plsc.load_gather(asrc_v, [s])
            a_d = plsc.load_gather(adst_v, [d])
            z = a_s + a_d
            lg = jnp.where(z >= 0.0, z, 0.2 * z)
            b = amax_t + a_d
            m = jnp.where(b >= 0.0, b, 0.2 * b)
            w_v[pl.ds(off, L)] = jnp.exp(lg - m)
            srcoff_v[pl.ds(off, L)] = s + t_n
            return 0
        lax.fori_loop(0, E_PER // L, p1, 0)

        plsc.subcore_barrier()

        def dchunk(c, _):
            base = pl.multiple_of(c * CH, CH)
            for v in range(CH // L):
                dstc_v[pl.ds(v * L, L)] = dst_v[pl.ds(base + v * L, L)]
            pltpu.sync_copy(w_v.at[pl.ds(base, CH)], den_sh.at[dstc_v],
                            add=True)
            return 0
        lax.fori_loop(0, NCH, dchunk, 0)

        plsc.subcore_barrier()

        pltpu.sync_copy(den_sh, den_v)

        def p2(i, _):
            off = pl.multiple_of(i * L, L)
            d = dst_v[pl.ds(off, L)]
            dn = plsc.load_gather(den_v, [d])
            w_v[pl.ds(off, L)] = (w_v[pl.ds(off, L)] * ea_v[pl.ds(off, L)]
                                  / (dn + 1e-16))
            return 0
        lax.fori_loop(0, E_PER // L, p2, 0)

        xmh = xm_hbm.at[cid]

        def start_gather(c, rows, sm):
            base = pl.multiple_of(c * CH, CH)
            pltpu.async_copy(xmh.at[srcoff_v.at[pl.ds(base, CH)]], rows, sm)

        def wait_gather(c, rows, sm):
            base = pl.multiple_of(c * CH, CH)
            pltpu.make_async_copy(
                xmh.at[srcoff_v.at[pl.ds(base, CH)]], rows, sm).wait()

        def proc(c, rows, dstc):
            base = pl.multiple_of(c * CH, CH)
            for v in range(CH // L):
                dstc[pl.ds(v * L, L)] = dst_v[pl.ds(base + v * L, L)]

            def srow(r4, _):
                r = r4 * 4
                for u in range(4):
                    rr = r + u
                    cs = plsc.load_gather(
                        w_v, [jnp.zeros((L,), jnp.int32) + (base + rr)])
                    for j in range(C2 // L):
                        rows[rr, pl.ds(j * L, L)] = (
                            rows[rr, pl.ds(j * L, L)] * cs)
                return 0
            lax.fori_loop(0, CH // 4, srow, 0)

            pltpu.sync_copy(rows, agg_sh.at[dstc], add=True)

        start_gather(0, rows_a, sem_a)

        def pair(i, _):
            ca = 2 * i
            cb = 2 * i + 1
            start_gather(cb, rows_b, sem_b)
            wait_gather(ca, rows_a, sem_a)
            proc(ca, rows_a, dstc_a)
            start_gather(ca + 2, rows_a, sem_a)
            wait_gather(cb, rows_b, sem_b)
            proc(cb, rows_b, dstc_b)
            return 0
        lax.fori_loop(0, NCH // 2, pair, 0)

        wait_gather(NCH - 1, rows_a, sem_a)
        proc(NCH - 1, rows_a, dstc_a)

        plsc.subcore_barrier()

        pltpu.sync_copy(agg_sh.at[pl.ds(nlo, ROWS_PER)],
                        out_hbm.at[cid].at[t].at[pl.ds(nlo, ROWS_PER)])


def _make_sc_kernel():
  return functools.partial(
    pl.kernel,
    out_type=jax.ShapeDtypeStruct((NC, T, NROW, C2), jnp.float32),
    mesh=plsc.VectorSubcoreMesh(core_axis_name="c", subcore_axis_name="s",
                                num_cores=NC, num_subcores=NS),
    compiler_params=pltpu.CompilerParams(needs_layout_passes=False,
                                         use_tc_tiling_on_sc=False),
    scratch_types=[
        pltpu.VMEM((E_PER,), jnp.int32),
        pltpu.VMEM((E_PER,), jnp.int32),
        pltpu.VMEM((E_PER,), jnp.float32),
        pltpu.VMEM((N,), jnp.float32),
        pltpu.VMEM((N,), jnp.float32),
        pltpu.VMEM((E_PER,), jnp.float32),
        pltpu.VMEM((NPAD,), jnp.float32),
        pltpu.VMEM((CH, C2), jnp.float32),
        pltpu.VMEM((CH, C2), jnp.float32),
        pltpu.VMEM((CH,), jnp.int32),
        pltpu.VMEM((CH,), jnp.int32),
        pltpu.VMEM((L,), jnp.float32),
        pltpu.VMEM_SHARED((NPAD,), jnp.float32),
        pltpu.VMEM_SHARED((NROW, C2), jnp.float32),
        pltpu.SemaphoreType.DMA,
        pltpu.SemaphoreType.DMA,
    ],
  )(_sc_body)


@jax.jit
def kernel(x, edge_index, edge_attr, W_tconv, b_tconv, W_msg, b_msg,
           W_self, b_self, W_asrc, b_asrc, W_adst, b_adst):
    x = x.astype(jnp.float32)
    src = edge_index[0].astype(jnp.int32)
    dst = edge_index[1].astype(jnp.int32)
    ea = edge_attr[:, 0].astype(jnp.float32)

    wt = jnp.transpose(W_tconv, (2, 1, 0))
    wm = W_msg.T
    ws = W_self.T
    bt = b_tconv.reshape(1, C)
    bm = b_msg.reshape(1, C)
    bs = b_self.reshape(1, C)
    bias = (b_asrc + b_adst).reshape(1, 1)

    xm, asrc, adst, amax = pl.pallas_call(
        _tc1_body,
        grid=(G1,),
        in_specs=[
            pl.BlockSpec((NB1, T, C), lambda i: (i, 0, 0)),
            pl.BlockSpec((K, C, C), lambda i: (0, 0, 0)),
            pl.BlockSpec((C, C), lambda i: (0, 0)),
            pl.BlockSpec((1, C), lambda i: (0, 0)),
            pl.BlockSpec((1, C), lambda i: (0, 0)),
            pl.BlockSpec((1, C), lambda i: (0, 0)),
            pl.BlockSpec((1, C), lambda i: (0, 0)),
            pl.BlockSpec((1, 1), lambda i: (0, 0)),
        ],
        out_specs=[
            pl.BlockSpec((NC, NB1, T, C2), lambda i: (0, i, 0, 0)),
            pl.BlockSpec((NB1, T), lambda i: (i, 0)),
            pl.BlockSpec((NB1, T), lambda i: (i, 0)),
            pl.BlockSpec((1, T), lambda i: (0, 0)),
        ],
        out_shape=[
            jax.ShapeDtypeStruct((NC, N, T, C2), jnp.float32),
            jax.ShapeDtypeStruct((N, T), jnp.float32),
            jax.ShapeDtypeStruct((N, T), jnp.float32),
            jax.ShapeDtypeStruct((1, T), jnp.float32),
        ],
    )(x, wt, wm, W_asrc, W_adst, bt, bm, bias)

    amax16 = jnp.pad(amax.reshape(T), (0, L - T))
    xmflat = xm.reshape(NC, N * T, C2)
    asrcflat = asrc.reshape(N * T)
    adstflat = adst.reshape(N * T)

    agg = _make_sc_kernel()(src, dst, ea, asrcflat, adstflat, amax16, xmflat)

    out = pl.pallas_call(
        _tc2_body,
        grid=(G2,),
        in_specs=[
            pl.BlockSpec((NB2, T, C), lambda i: (i, 0, 0)),
            pl.BlockSpec((NC, 1, NROW, C2), lambda i: (0, i, 0, 0)),
            pl.BlockSpec((C, C), lambda i: (0, 0)),
            pl.BlockSpec((1, C), lambda i: (0, 0)),
        ],
        out_specs=pl.BlockSpec((NB2, T, C), lambda i: (i, 0, 0)),
        out_shape=jax.ShapeDtypeStruct((N, T, C), jnp.float32),
        compiler_params=pltpu.CompilerParams(
            vmem_limit_bytes=64 * 1024 * 1024),
    )(x, agg, ws, bs)

    return out
